# baseline (speedup 1.0000x reference)
import jax
import jax.numpy as jnp
from jax import lax
from jax.experimental import pallas as pl
from jax.experimental.pallas import tpu as pltpu
from jax.experimental.pallas import tpu_sc as plsc

B = 1024
S = 50
NW = 32
NB_W = B // NW
LANES = 16
ROWS_B = S * S
REL_CHUNKS = ((0, 1248), (1248, 1248), (2496, 1248), (3744, 1256))
CMAX = 1256
J_CHUNKS = (0, 16, 32, 34)


def _splat(v):
    return jnp.full((LANES,), v, jnp.int32)


def _body(x_hbm, ct_hbm, pt_hbm,
          pos_hbm, rel_hbm,
          x_v, codes_v, pcodes_v,
          rel_bufs0, rel_bufs1, pos_buf,
          sem_g, sem_r0, sem_r1, sem_pg, sem_po):
    cid = lax.axis_index("c")
    sid = lax.axis_index("s")
    wid = sid * 2 + cid
    b0 = wid * NB_W

    pltpu.sync_copy(x_hbm.at[pl.ds(b0 * 2 * S, NB_W * 2 * S)], x_v)

    rel_bufs = (rel_bufs0, rel_bufs1)
    rel_sems = (sem_r0, sem_r1)

    def _codes_batch(bl, cbase, pbase):
        xoff = bl * 2 * S
        xj0 = [x_v[pl.ds(xoff + c, LANES)] for c in J_CHUNKS]
        xj1 = [x_v[pl.ds(xoff + S + c, LANES)] for c in J_CHUNKS]

        for ci, c in enumerate(J_CHUNKS):
            i0 = jnp.minimum(jnp.abs(xj0[ci]), 15)
            i1 = jnp.minimum(jnp.abs(xj1[ci]), 15)
            pcodes_v[pl.ds(pbase + c, LANES)] = (i0 << 4) | i1

        def i_row(il, carry):
            xi0 = plsc.load_gather(x_v, [_splat(xoff + il)])
            xi1 = plsc.load_gather(x_v, [_splat(xoff + S + il)])
            obase = cbase + il * S
            for ci, c in enumerate(J_CHUNKS):
                a = jnp.minimum(jnp.abs(xj0[ci] - xi0), 15)
                p = jnp.minimum(jnp.abs(xj1[ci] - xi1), 3)
                codes_v[pl.ds(obase + c, LANES)] = (a << 2) | p
            return carry

        lax.fori_loop(0, S, i_row, 0)

    def group(g, carry):
        for pr in range(2):
            pb = g * 4 + pr * 2
            pair_row0 = (b0 + pb) * ROWS_B
            _codes_batch(pb, 0, (pr * 2) * S)
            _codes_batch(pb + 1, ROWS_B, (pr * 2 + 1) * S)
            for kc, (off, cnt) in enumerate(REL_CHUNKS):
                sl = kc % 2
                buf = rel_bufs[sl].at[pl.ds(0, cnt)]
                cnt_prev = REL_CHUNKS[kc - 2][1] if kc >= 2 else (
                    1248 if sl == 0 else 1256)
                prev_wait = pltpu.make_async_copy(
                    rel_bufs[sl].at[pl.ds(0, cnt_prev)],
                    rel_hbm.at[pl.ds(pl.multiple_of(pair_row0, 8), cnt_prev)],
                    rel_sems[sl])
                if kc >= 2 or pr == 1:
                    prev_wait.wait()
                else:
                    @pl.when(g > 0)
                    def _w():
                        prev_wait.wait()
                gchunks = [(o2, min(96, cnt - o2)) for o2 in range(0, cnt, 96)]
                gcopies = [
                    pltpu.make_async_copy(
                        ct_hbm.at[codes_v.at[pl.ds(off + o2, c2)]],
                        rel_bufs[sl].at[pl.ds(o2, c2)], sem_g)
                    for o2, c2 in gchunks]
                for gc in gcopies:
                    gc.start()
                for gc in gcopies:
                    gc.wait()
                pltpu.make_async_copy(
                    buf,
                    rel_hbm.at[pl.ds(pl.multiple_of(pair_row0 + off, 8), cnt)],
                    rel_sems[sl]).start()

        pos_row0 = (b0 + g * 4) * S
        pos_out = pos_hbm.at[pl.ds(pl.multiple_of(pos_row0, 8), 4 * S)]
        pos_wait = pltpu.make_async_copy(pos_buf, pos_out, sem_po)

        @pl.when(g > 0)
        def _wp():
            pos_wait.wait()

        pchunks = [(0, 96), (96, 96), (192, 8)]
        pcopies = [
            pltpu.make_async_copy(
                pt_hbm.at[pcodes_v.at[pl.ds(o2, c2)]],
                pos_buf.at[pl.ds(o2, c2)], sem_pg)
            for o2, c2 in pchunks]
        for pc in pcopies:
            pc.start()
        for pc in pcopies:
            pc.wait()
        pltpu.make_async_copy(pos_buf, pos_out, sem_po).start()
        return carry

    lax.fori_loop(0, NB_W // 4, group, 0)

    last_row0 = (b0 + NB_W - 2) * ROWS_B
    pltpu.make_async_copy(
        rel_bufs[0].at[pl.ds(0, 1248)],
        rel_hbm.at[pl.ds(pl.multiple_of(last_row0, 8), 1248)],
        sem_r0).wait()
    pltpu.make_async_copy(
        rel_bufs[1].at[pl.ds(0, 1256)],
        rel_hbm.at[pl.ds(pl.multiple_of(last_row0 + 3744, 8), 1256)],
        sem_r1).wait()
    pltpu.make_async_copy(
        pos_buf,
        pos_hbm.at[pl.ds(pl.multiple_of((b0 + NB_W - 4) * S, 8), 4 * S)],
        sem_po).wait()


@jax.jit
def _sc_position_layer(x, ct, pt):
    mesh = plsc.VectorSubcoreMesh(core_axis_name="c", subcore_axis_name="s")
    f = pl.kernel(
        _body,
        out_type=(jax.ShapeDtypeStruct((B * S, 32), jnp.float32),
                  jax.ShapeDtypeStruct((B * ROWS_B, 32), jnp.float32)),
        mesh=mesh,
        scratch_types=[
            pltpu.VMEM((NB_W * 2 * S,), jnp.int32),
            pltpu.VMEM((2 * ROWS_B + 8,), jnp.int32),
            pltpu.VMEM((208,), jnp.int32),
            pltpu.VMEM((CMAX, 32), jnp.float32),
            pltpu.VMEM((CMAX, 32), jnp.float32),
            pltpu.VMEM((4 * S, 32), jnp.float32),
            pltpu.SemaphoreType.DMA,
            pltpu.SemaphoreType.DMA,
            pltpu.SemaphoreType.DMA,
            pltpu.SemaphoreType.DMA,
            pltpu.SemaphoreType.DMA,
        ],
        compiler_params=pltpu.CompilerParams(needs_layout_passes=False,
                                             use_tc_tiling_on_sc=False),
    )
    return f(x, ct, pt)


def kernel(x_position_info, pos_post_emb, pos_para_emb, dist_post_emb, dist_para_emb):
    x = x_position_info.astype(jnp.int32).transpose(0, 2, 1).reshape(B * 2 * S)
    ct = jnp.concatenate([jnp.repeat(dist_para_emb, 4, axis=0),
                          jnp.tile(dist_post_emb, (16, 1))], axis=1)
    pt = jnp.concatenate([jnp.repeat(pos_post_emb, 16, axis=0),
                          jnp.tile(pos_para_emb, (16, 1))], axis=1)
    pos_flat, rel_flat = _sc_position_layer(x, ct, pt)
    return (pos_flat.reshape(B, S, 32), rel_flat.reshape(B, S, S, 32))


# speedup vs baseline: 7.0747x; 7.0747x over previous
import jax
import jax.numpy as jnp
from jax import lax
from jax.experimental import pallas as pl
from jax.experimental.pallas import tpu as pltpu
from jax.experimental.pallas import tpu_sc as plsc

B = 1024
S = 50
HALF = 25
NW = 32
NB_W = B // NW
LANES = 16
REL_ROW = S * 32
REL_HB = HALF * REL_ROW
POS_ROW = S * 32
CHUNK_LANES = ((0, tuple(range(16))), (16, tuple(range(16))),
               (32, tuple(range(16))), (34, (14, 15)))


def _splat(v):
    return jnp.full((LANES,), v, jnp.int32)


_LSPLATS = None


def _bcast(vec, l):
    return jnp.take_along_axis(vec, _LSPLATS[l], axis=0,
                               mode="promise_in_bounds")


def _emit_rows(dst, obase, tab_a, tab_b, a16, b16, lanes, c, iota):
    for gbase in range(0, len(lanes), 4):
        grp = lanes[gbase:gbase + 4]
        idxs = [(_bcast(a16, l) | iota, _bcast(b16, l) | iota) for l in grp]
        rows = [(plsc.load_gather(tab_a, [ia]), plsc.load_gather(tab_b, [ib]))
                for ia, ib in idxs]
        for l, (ra, rb) in zip(grp, rows):
            o = obase + (c + l) * 32
            dst[pl.ds(o, 16)] = ra
            dst[pl.ds(o + 16, 16)] = rb


def _body(x_hbm, ppost_hbm, ppara_hbm, dpost_hbm, dpara_hbm,
          pos_hbm, rel_hbm,
          x_v, ppost_v, ppara_v, dpost_v, dpara_v,
          rel_b0, rel_b1, pos_b0, pos_b1,
          sem_r0, sem_r1, sem_p0, sem_p1):
    global _LSPLATS
    _LSPLATS = [_splat(l) for l in range(LANES)]
    wid = lax.axis_index("s") * 2 + lax.axis_index("c")
    b0 = wid * NB_W

    pltpu.sync_copy(x_hbm.at[pl.ds(b0 * 2 * S, NB_W * 2 * S)], x_v)
    pltpu.sync_copy(ppost_hbm, ppost_v)
    pltpu.sync_copy(ppara_hbm, ppara_v)
    pltpu.sync_copy(dpost_hbm, dpost_v)
    pltpu.sync_copy(dpara_hbm, dpara_v)

    iota = lax.iota(jnp.int32, LANES)
    rel_bufs = (rel_b0, rel_b1)
    rel_sems = (sem_r0, sem_r1)
    pos_bufs = (pos_b0, pos_b1)
    pos_sems = (sem_p0, sem_p1)

    def batch_pair(bp, carry):
        for b_par in range(2):
            bl = bp * 2 + b_par
            b = b0 + bl
            xoff = bl * 2 * S
            xj0 = [x_v[pl.ds(xoff + c, LANES)] for c, _ in CHUNK_LANES]
            xj1 = [x_v[pl.ds(xoff + S + c, LANES)] for c, _ in CHUNK_LANES]

            pos_buf = pos_bufs[b_par]
            psem = pos_sems[b_par]
            pos_dst = pos_hbm.at[pl.ds(b * POS_ROW, POS_ROW)]

            @pl.when(bp > 0)
            def _wait_pos():
                pltpu.make_async_copy(pos_buf, pos_dst, psem).wait()

            for ci, (c, lanes) in enumerate(CHUNK_LANES):
                i0 = jnp.minimum(jnp.abs(xj0[ci]), 15) << 4
                i1 = jnp.minimum(jnp.abs(xj1[ci]), 15) << 4
                _emit_rows(pos_buf, 0, ppost_v, ppara_v, i0, i1,
                           lanes, c, iota)
            pltpu.make_async_copy(pos_buf, pos_dst, psem).start()

            for h in range(2):
                rel_buf = rel_bufs[h]
                rsem = rel_sems[h]
                dst = rel_hbm.at[pl.ds(b * 2 * REL_HB + h * REL_HB, REL_HB)]

                @pl.when(bl > 0)
                def _wait_rel():
                    pltpu.make_async_copy(rel_buf, dst, rsem).wait()

                def i_row(il, c2):
                    ig = h * HALF + il
                    xi0 = plsc.load_gather(x_v, [_splat(xoff + ig)])
                    xi1 = plsc.load_gather(x_v, [_splat(xoff + S + ig)])
                    obase = il * REL_ROW
                    for ci, (c, lanes) in enumerate(CHUNK_LANES):
                        a16 = jnp.minimum(jnp.abs(xj0[ci] - xi0), 15) << 4
                        p16 = jnp.minimum(jnp.abs(xj1[ci] - xi1), 3) << 4
                        _emit_rows(rel_buf, obase, dpara_v, dpost_v,
                                   a16, p16, lanes, c, iota)
                    return c2

                lax.fori_loop(0, HALF, i_row, 0)
                pltpu.make_async_copy(rel_buf, dst, rsem).start()
        return carry

    lax.fori_loop(0, NB_W // 2, batch_pair, 0)

    b_last = b0 + NB_W - 1
    for h in range(2):
        pltpu.make_async_copy(
            rel_bufs[h],
            rel_hbm.at[pl.ds(b_last * 2 * REL_HB + h * REL_HB, REL_HB)],
            rel_sems[h]).wait()
    for b_par in range(2):
        pltpu.make_async_copy(pos_bufs[b_par],
                              pos_hbm.at[pl.ds(b_last * POS_ROW, POS_ROW)],
                              pos_sems[b_par]).wait()


@jax.jit
def _sc_position_layer(x, ppost, ppara, dpost, dpara):
    mesh = plsc.VectorSubcoreMesh(core_axis_name="c", subcore_axis_name="s")
    f = pl.kernel(
        _body,
        out_type=(jax.ShapeDtypeStruct((B * POS_ROW,), jnp.float32),
                  jax.ShapeDtypeStruct((B * 2 * REL_HB,), jnp.float32)),
        mesh=mesh,
        scratch_types=[
            pltpu.VMEM((NB_W * 2 * S,), jnp.int32),
            pltpu.VMEM((256,), jnp.float32),
            pltpu.VMEM((256,), jnp.float32),
            pltpu.VMEM((64,), jnp.float32),
            pltpu.VMEM((256,), jnp.float32),
            pltpu.VMEM((REL_HB,), jnp.float32),
            pltpu.VMEM((REL_HB,), jnp.float32),
            pltpu.VMEM((POS_ROW,), jnp.float32),
            pltpu.VMEM((POS_ROW,), jnp.float32),
            pltpu.SemaphoreType.DMA,
            pltpu.SemaphoreType.DMA,
            pltpu.SemaphoreType.DMA,
            pltpu.SemaphoreType.DMA,
        ],
        compiler_params=pltpu.CompilerParams(needs_layout_passes=False,
                                             use_tc_tiling_on_sc=False),
    )
    return f(x, ppost, ppara, dpost, dpara)


def kernel(x_position_info, pos_post_emb, pos_para_emb, dist_post_emb, dist_para_emb):
    x = x_position_info.astype(jnp.int32).transpose(0, 2, 1).reshape(B * 2 * S)
    pos_flat, rel_flat = _sc_position_layer(
        x, pos_post_emb.reshape(-1), pos_para_emb.reshape(-1),
        dist_post_emb.reshape(-1), dist_para_emb.reshape(-1))
    return (pos_flat.reshape(B, S, 32), rel_flat.reshape(B, S, S, 32))
